# baseline (speedup 1.0000x reference)
import functools

import jax
import jax.numpy as jnp
from jax.experimental import pallas as pl
from jax.experimental.pallas import tpu as pltpu


def _lstm_chunk_kernel(x_ref, m_ref, wih_ref, whh_ref, b_ref,
                       y_ref, hT_ref, cT_ref,
                       g_s, h_s, c_s, *, ts, hidden):
    @pl.when(pl.program_id(0) == 0)
    def _init():
        h_s[...] = jnp.zeros_like(h_s)
        c_s[...] = jnp.zeros_like(c_s)

    g_s[...] = jax.lax.dot_general(
        x_ref[...], wih_ref[...], (((2,), (0,)), ((), ())),
        preferred_element_type=jnp.float32)

    bias = b_ref[...]

    def step(t, carry):
        h = h_s[...]
        c = c_s[...]
        gates = g_s[t] + jnp.dot(h.astype(jnp.bfloat16), whh_ref[...],
                                 preferred_element_type=jnp.float32)
        gates = gates + bias
        i_g = jax.nn.sigmoid(gates[:, 0 * hidden:1 * hidden])
        f_g = jax.nn.sigmoid(gates[:, 1 * hidden:2 * hidden])
        g_g = jnp.tanh(gates[:, 2 * hidden:3 * hidden])
        o_g = jax.nn.sigmoid(gates[:, 3 * hidden:4 * hidden])
        c_new = f_g * c + i_g * g_g
        h_new = o_g * jnp.tanh(c_new)
        valid = m_ref[t] > 0.0
        h_s[...] = jnp.where(valid, h_new, h)
        c_s[...] = jnp.where(valid, c_new, c)
        y_ref[t] = jnp.where(valid, h_new, jnp.zeros_like(h_new))
        return carry

    jax.lax.fori_loop(0, ts, step, 0, unroll=False)

    hT_ref[...] = h_s[...]
    cT_ref[...] = c_s[...]


def kernel(inputs, mask, W_ih, W_hh, b):
    B, T, D = inputs.shape
    H = W_hh.shape[0]
    TS = 128
    num_chunks = T // TS

    x_tm = jnp.swapaxes(inputs, 0, 1)
    m_tm = jnp.swapaxes(mask, 0, 1).astype(jnp.float32)[:, :, None]
    b2 = b.reshape(1, 4 * H)

    grid_spec = pltpu.PrefetchScalarGridSpec(
        num_scalar_prefetch=0,
        grid=(num_chunks,),
        in_specs=[
            pl.BlockSpec((TS, B, D), lambda i: (i, 0, 0)),
            pl.BlockSpec((TS, B, 1), lambda i: (i, 0, 0)),
            pl.BlockSpec((D, 4 * H), lambda i: (0, 0)),
            pl.BlockSpec((H, 4 * H), lambda i: (0, 0)),
            pl.BlockSpec((1, 4 * H), lambda i: (0, 0)),
        ],
        out_specs=[
            pl.BlockSpec((TS, B, H), lambda i: (i, 0, 0)),
            pl.BlockSpec((B, H), lambda i: (0, 0)),
            pl.BlockSpec((B, H), lambda i: (0, 0)),
        ],
        scratch_shapes=[
            pltpu.VMEM((TS, B, 4 * H), jnp.float32),
            pltpu.VMEM((B, H), jnp.float32),
            pltpu.VMEM((B, H), jnp.float32),
        ],
    )

    y_tm, hT, cT = pl.pallas_call(
        functools.partial(_lstm_chunk_kernel, ts=TS, hidden=H),
        grid_spec=grid_spec,
        out_shape=[
            jax.ShapeDtypeStruct((T, B, H), jnp.float32),
            jax.ShapeDtypeStruct((B, H), jnp.float32),
            jax.ShapeDtypeStruct((B, H), jnp.float32),
        ],
        compiler_params=pltpu.CompilerParams(
            dimension_semantics=("arbitrary",),
        ),
    )(x_tm, m_tm, W_ih, W_hh.astype(jnp.bfloat16), b2)

    outputs = jnp.swapaxes(y_tm, 0, 1)
    return outputs, hT, cT


# speedup vs baseline: 1.0387x; 1.0387x over previous
import functools

import jax
import jax.numpy as jnp
from jax.experimental import pallas as pl
from jax.experimental.pallas import tpu as pltpu


def _lstm_chunk_kernel(x_ref, m_ref, wih_ref, whh_ref, b_ref,
                       y_ref, hT_ref, cT_ref,
                       g_s, h_s, c_s, *, ts, hidden):
    @pl.when(pl.program_id(0) == 0)
    def _init():
        h_s[...] = jnp.zeros_like(h_s)
        c_s[...] = jnp.zeros_like(c_s)

    g_s[...] = jax.lax.dot_general(
        x_ref[...], wih_ref[...], (((2,), (0,)), ((), ())),
        preferred_element_type=jnp.float32)

    bias = b_ref[...]

    def step(t, carry):
        h, c = carry
        gates = g_s[t] + jnp.dot(h.astype(jnp.bfloat16), whh_ref[...],
                                 preferred_element_type=jnp.float32)
        gates = gates + bias
        i_g = jax.nn.sigmoid(gates[:, 0 * hidden:1 * hidden])
        f_g = jax.nn.sigmoid(gates[:, 1 * hidden:2 * hidden])
        g_g = jnp.tanh(gates[:, 2 * hidden:3 * hidden])
        o_g = jax.nn.sigmoid(gates[:, 3 * hidden:4 * hidden])
        c_new = f_g * c + i_g * g_g
        h_new = o_g * jnp.tanh(c_new)
        valid = m_ref[t] > 0.0
        h2 = jnp.where(valid, h_new, h)
        c2 = jnp.where(valid, c_new, c)
        y_ref[t] = jnp.where(valid, h_new, jnp.zeros_like(h_new))
        return (h2, c2)

    hT, cT = jax.lax.fori_loop(0, ts, step, (h_s[...], c_s[...]), unroll=2)
    h_s[...] = hT
    c_s[...] = cT

    hT_ref[...] = hT
    cT_ref[...] = cT


def kernel(inputs, mask, W_ih, W_hh, b):
    B, T, D = inputs.shape
    H = W_hh.shape[0]
    TS = 128
    num_chunks = T // TS

    x_tm = jnp.swapaxes(inputs, 0, 1)
    m_tm = jnp.swapaxes(mask, 0, 1).astype(jnp.float32)[:, :, None]
    b2 = b.reshape(1, 4 * H)

    grid_spec = pltpu.PrefetchScalarGridSpec(
        num_scalar_prefetch=0,
        grid=(num_chunks,),
        in_specs=[
            pl.BlockSpec((TS, B, D), lambda i: (i, 0, 0)),
            pl.BlockSpec((TS, B, 1), lambda i: (i, 0, 0)),
            pl.BlockSpec((D, 4 * H), lambda i: (0, 0)),
            pl.BlockSpec((H, 4 * H), lambda i: (0, 0)),
            pl.BlockSpec((1, 4 * H), lambda i: (0, 0)),
        ],
        out_specs=[
            pl.BlockSpec((TS, B, H), lambda i: (i, 0, 0)),
            pl.BlockSpec((B, H), lambda i: (0, 0)),
            pl.BlockSpec((B, H), lambda i: (0, 0)),
        ],
        scratch_shapes=[
            pltpu.VMEM((TS, B, 4 * H), jnp.float32),
            pltpu.VMEM((B, H), jnp.float32),
            pltpu.VMEM((B, H), jnp.float32),
        ],
    )

    y_tm, hT, cT = pl.pallas_call(
        functools.partial(_lstm_chunk_kernel, ts=TS, hidden=H),
        grid_spec=grid_spec,
        out_shape=[
            jax.ShapeDtypeStruct((T, B, H), jnp.float32),
            jax.ShapeDtypeStruct((B, H), jnp.float32),
            jax.ShapeDtypeStruct((B, H), jnp.float32),
        ],
        compiler_params=pltpu.CompilerParams(
            dimension_semantics=("arbitrary",),
        ),
    )(x_tm, m_tm, W_ih, W_hh.astype(jnp.bfloat16), b2)

    outputs = jnp.swapaxes(y_tm, 0, 1)
    return outputs, hT, cT


# speedup vs baseline: 1.0796x; 1.0394x over previous
import functools

import jax
import jax.numpy as jnp
from jax.experimental import pallas as pl
from jax.experimental.pallas import tpu as pltpu


def _lstm_chunk_kernel(x_ref, m_ref, wih_ref, whh_ref, b_ref,
                       y_ref, hT_ref, cT_ref,
                       g_s, h_s, c_s, *, ts, hidden):
    @pl.when(pl.program_id(0) == 0)
    def _init():
        h_s[...] = jnp.zeros_like(h_s)
        c_s[...] = jnp.zeros_like(c_s)

    g_s[...] = jax.lax.dot_general(
        x_ref[...], wih_ref[...], (((2,), (0,)), ((), ())),
        preferred_element_type=jnp.float32)

    bias = b_ref[...]

    def step(t, carry):
        h, c = carry
        gates = g_s[t] + jnp.dot(h.astype(jnp.bfloat16), whh_ref[...],
                                 preferred_element_type=jnp.float32)
        gates = gates + bias
        i_g = jax.nn.sigmoid(gates[:, 0 * hidden:1 * hidden])
        f_g = jax.nn.sigmoid(gates[:, 1 * hidden:2 * hidden])
        g_g = jnp.tanh(gates[:, 2 * hidden:3 * hidden])
        o_g = jax.nn.sigmoid(gates[:, 3 * hidden:4 * hidden])
        c_new = f_g * c + i_g * g_g
        h_new = o_g * jnp.tanh(c_new)
        valid = m_ref[t] > 0.0
        h2 = jnp.where(valid, h_new, h)
        c2 = jnp.where(valid, c_new, c)
        y_ref[:, pl.ds(t, 1), :] = jnp.where(valid, h_new,
                                             jnp.zeros_like(h_new))[:, None, :]
        return (h2, c2)

    hT, cT = jax.lax.fori_loop(0, ts, step, (h_s[...], c_s[...]), unroll=2)
    h_s[...] = hT
    c_s[...] = cT

    hT_ref[...] = hT
    cT_ref[...] = cT


def kernel(inputs, mask, W_ih, W_hh, b):
    B, T, D = inputs.shape
    H = W_hh.shape[0]
    TS = 128
    num_chunks = T // TS

    x_tm = jnp.swapaxes(inputs, 0, 1)
    m_tm = jnp.swapaxes(mask, 0, 1).astype(jnp.float32)[:, :, None]
    b2 = b.reshape(1, 4 * H)

    grid_spec = pltpu.PrefetchScalarGridSpec(
        num_scalar_prefetch=0,
        grid=(num_chunks,),
        in_specs=[
            pl.BlockSpec((TS, B, D), lambda i: (i, 0, 0)),
            pl.BlockSpec((TS, B, 1), lambda i: (i, 0, 0)),
            pl.BlockSpec((D, 4 * H), lambda i: (0, 0)),
            pl.BlockSpec((H, 4 * H), lambda i: (0, 0)),
            pl.BlockSpec((1, 4 * H), lambda i: (0, 0)),
        ],
        out_specs=[
            pl.BlockSpec((B, TS, H), lambda i: (0, i, 0)),
            pl.BlockSpec((B, H), lambda i: (0, 0)),
            pl.BlockSpec((B, H), lambda i: (0, 0)),
        ],
        scratch_shapes=[
            pltpu.VMEM((TS, B, 4 * H), jnp.float32),
            pltpu.VMEM((B, H), jnp.float32),
            pltpu.VMEM((B, H), jnp.float32),
        ],
    )

    outputs, hT, cT = pl.pallas_call(
        functools.partial(_lstm_chunk_kernel, ts=TS, hidden=H),
        grid_spec=grid_spec,
        out_shape=[
            jax.ShapeDtypeStruct((B, T, H), jnp.float32),
            jax.ShapeDtypeStruct((B, H), jnp.float32),
            jax.ShapeDtypeStruct((B, H), jnp.float32),
        ],
        compiler_params=pltpu.CompilerParams(
            dimension_semantics=("arbitrary",),
        ),
    )(x_tm, m_tm, W_ih, W_hh.astype(jnp.bfloat16), b2)

    return outputs, hT, cT
